# baseline (speedup 1.0000x reference)
import functools

import jax
import jax.numpy as jnp
from jax import lax
from jax.experimental import pallas as pl
from jax.experimental.pallas import tpu as pltpu
from jax.experimental.pallas import tpu_sc as plsc

D = 128
CH = 128
GP = 8
NC = 2
NS = 16
NW = NC * NS
ACC_ROWS = 10240
NSLOT = 2


def _sc_aggregate(table, src_g, dst_g, zeros_blk):
    g_total = src_g.shape[0]
    t_per = g_total // NW
    assert t_per % 2 == 0 and g_total % NW == 0
    TT = t_per // 2
    zrows = ACC_ROWS // NS

    mesh = plsc.VectorSubcoreMesh(
        core_axis_name="c", subcore_axis_name="s",
        num_cores=NC, num_subcores=NS)

    @functools.partial(
        pl.kernel,
        out_type=jax.ShapeDtypeStruct((NC, ACC_ROWS, D), jnp.float32),
        mesh=mesh,
        scratch_types=[
            pltpu.VMEM_SHARED((ACC_ROWS, D), jnp.float32),
            pltpu.VMEM((NSLOT, CH, D), jnp.float32),
            pltpu.VMEM((2, GP, CH), jnp.int32),
            pltpu.VMEM((2, GP, CH), jnp.int32),
            pltpu.SemaphoreType.DMA,
            pltpu.SemaphoreType.DMA,
            pltpu.SemaphoreType.DMA,
            pltpu.SemaphoreType.DMA,
            pltpu.SemaphoreType.DMA,
            pltpu.SemaphoreType.DMA,
        ],
    )
    def agg(table_hbm, src_hbm, dst_hbm, zeros_hbm, out_hbm,
            acc, rows, sib, dib,
            g0, g1, s0, s1, w0, w1):
        gsem = (g0, g1)
        ssem = (s0, s1)
        wsem = (w0, w1)
        cid = lax.axis_index("c")
        sid = lax.axis_index("s")
        wid = sid * NC + cid

        pltpu.sync_copy(zeros_hbm, rows.at[0])
        for k in range(zrows // CH):
            pltpu.async_copy(
                rows.at[0], acc.at[pl.ds(sid * zrows + k * CH, CH)], w0)
        for k in range(zrows // CH):
            pltpu.make_async_copy(
                rows.at[0], acc.at[pl.ds(sid * zrows + k * CH, CH)], w0).wait()
        plsc.subcore_barrier()

        def g_start(par, k, slot):
            pltpu.async_copy(table_hbm.at[sib.at[par, k]],
                             rows.at[slot], gsem[slot])

        def g_wait(par, k, slot):
            pltpu.make_async_copy(table_hbm.at[sib.at[par, k]],
                                  rows.at[slot], gsem[slot]).wait()

        def s_start(par, k, slot):
            pltpu.async_copy(rows.at[slot], acc.at[dib.at[par, k]],
                             ssem[slot], add=True)

        def s_wait(par, k, slot):
            pltpu.make_async_copy(rows.at[slot], acc.at[dib.at[par, k]],
                                  ssem[slot]).wait()

        def load_idx(par, t):
            g = wid + t * NW
            pltpu.sync_copy(src_hbm.at[g], sib.at[par])
            pltpu.sync_copy(dst_hbm.at[g], dib.at[par])

        load_idx(0, 0)
        g_start(0, 0, 0)

        def body(tt, carry):
            for half in (0, 1):
                for k in range(GP):
                    slot = k % NSLOT
                    o = 1 - slot
                    g_wait(half, k, slot)
                    s_start(half, k, slot)
                    if k == 5:
                        if half == 0:
                            load_idx(1, 2 * tt + 1)
                        else:
                            @pl.when(tt < TT - 1)
                            def _():
                                load_idx(0, 2 * tt + 2)
                    if k == 0:
                        if half == 0:
                            @pl.when(tt > 0)
                            def _():
                                s_wait(1, GP - 1, o)
                        else:
                            s_wait(0, GP - 1, o)
                    else:
                        s_wait(half, k - 1, o)
                    if k < GP - 1:
                        g_start(half, k + 1, o)
                    elif half == 0:
                        g_start(1, 0, o)
                    else:
                        @pl.when(tt < TT - 1)
                        def _():
                            g_start(0, 0, o)
            return carry

        lax.fori_loop(0, TT, body, 0)
        s_wait(1, GP - 1, (GP - 1) % NSLOT)
        plsc.subcore_barrier()

        nwb = zrows // CH
        for k in range(nwb):
            s = k % 2
            r0 = sid * zrows + k * CH
            if k >= 2:
                rp = sid * zrows + (k - 2) * CH
                pltpu.make_async_copy(
                    rows.at[s], out_hbm.at[cid, pl.ds(rp, CH)], wsem[s]).wait()
            pltpu.sync_copy(acc.at[pl.ds(r0, CH)], rows.at[s])
            pltpu.async_copy(rows.at[s], out_hbm.at[cid, pl.ds(r0, CH)], wsem[s])
        for k in range(nwb - 2, nwb):
            s = k % 2
            r0 = sid * zrows + k * CH
            pltpu.make_async_copy(
                rows.at[s], out_hbm.at[cid, pl.ds(r0, CH)], wsem[s]).wait()

    return agg(table, src_g, dst_g, zeros_blk)


def _tc_layer(agg2, hprev, wt, wlt, bias, apply_elu, n):
    bn = 1000
    dout = wt.shape[1]

    def body(p0_r, p1_r, hp_r, wt_r, wlt_r, b_r, o_r):
        aggm = p0_r[0] + p1_r[0]
        y = jnp.dot(aggm, wt_r[...], preferred_element_type=jnp.float32)
        y = y + jnp.dot(hp_r[...], wlt_r[...], preferred_element_type=jnp.float32)
        y = y + b_r[...]
        if apply_elu:
            y = jnp.where(y > 0, y, jnp.exp(jnp.minimum(y, 0.0)) - 1.0)
        o_r[...] = y

    return pl.pallas_call(
        body,
        grid=(n // bn,),
        in_specs=[
            pl.BlockSpec((1, bn, D), lambda i: (0, i, 0)),
            pl.BlockSpec((1, bn, D), lambda i: (1, i, 0)),
            pl.BlockSpec((bn, D), lambda i: (i, 0)),
            pl.BlockSpec((D, dout), lambda i: (0, 0)),
            pl.BlockSpec((D, dout), lambda i: (0, 0)),
            pl.BlockSpec((1, dout), lambda i: (0, 0)),
        ],
        out_specs=pl.BlockSpec((bn, dout), lambda i: (i, 0)),
        out_shape=jax.ShapeDtypeStruct((n, dout), jnp.float32),
    )(agg2, agg2, hprev, wt, wlt, bias)


def kernel(x, edge_index, W1, b1, W2, b2, W3, b3,
           Wl1, bl1, Wl2, bl2, Wl3, bl3):
    n = x.shape[0]
    e = edge_index.shape[1]
    gsz = GP * CH
    g_total = -(-e // gsz)
    g_total += (-g_total) % NW
    e_pad = g_total * gsz - e
    src_g = jnp.concatenate(
        [edge_index[0], jnp.zeros((e_pad,), jnp.int32)]).reshape(g_total, GP, CH)
    dst_g = jnp.concatenate(
        [edge_index[1], jnp.full((e_pad,), n, jnp.int32)]).reshape(g_total, GP, CH)
    zeros_blk = jnp.zeros((CH, D), jnp.float32)

    agg1 = _sc_aggregate(x, src_g, dst_g, zeros_blk)
    h1 = _tc_layer(agg1, x, W1.T, Wl1.T, (b1 + bl1)[None, :], True, n)
    agg2 = _sc_aggregate(h1, src_g, dst_g, zeros_blk)
    h2 = _tc_layer(agg2, h1, W2.T, Wl2.T, (b2 + bl2)[None, :], True, n)
    agg3 = _sc_aggregate(h2, src_g, dst_g, zeros_blk)
    w3p = jnp.pad(W3, ((0, D - W3.shape[0]), (0, 0)))
    wl3p = jnp.pad(Wl3, ((0, D - Wl3.shape[0]), (0, 0)))
    b3p = jnp.pad(b3 + bl3, (0, D - b3.shape[0]))
    out = _tc_layer(agg3, h2, w3p.T, wl3p.T, b3p[None, :], False, n)
    return out[:, :W3.shape[0]]


# speedup vs baseline: 2.9447x; 2.9447x over previous
import functools

import jax
import jax.numpy as jnp
from jax import lax
from jax.experimental import pallas as pl
from jax.experimental.pallas import tpu as pltpu
from jax.experimental.pallas import tpu_sc as plsc

D = 128
CH = 128
NC = 2
NS = 16
NW = NC * NS
ACC_ROWS = 10240


def _sc_aggregate(table, src_r, dst_r, zeros_blk):
    nch = src_r.shape[0] // CH
    base_trips, rem = divmod(nch, NW)
    zrows = ACC_ROWS // NS

    mesh = plsc.VectorSubcoreMesh(
        core_axis_name="c", subcore_axis_name="s",
        num_cores=NC, num_subcores=NS)

    @functools.partial(
        pl.kernel,
        out_type=jax.ShapeDtypeStruct((NC, ACC_ROWS, D), jnp.float32),
        mesh=mesh,
        scratch_types=[
            pltpu.VMEM_SHARED((ACC_ROWS, D), jnp.float32),
            pltpu.VMEM((2, CH, D), jnp.float32),
            pltpu.VMEM((4, CH), jnp.int32),
            pltpu.VMEM((4, CH), jnp.int32),
            pltpu.SemaphoreType.DMA,
            pltpu.SemaphoreType.DMA,
            pltpu.SemaphoreType.DMA,
            pltpu.SemaphoreType.DMA,
        ],
    )
    def agg(table_hbm, src_hbm, dst_hbm, zeros_hbm, out_hbm,
            acc, rows, sib, dib, isem, gsem, w0, w1):
        wsem = (w0, w1)
        cid = lax.axis_index("c")
        sid = lax.axis_index("s")
        wid = sid * NC + cid

        pltpu.sync_copy(zeros_hbm, rows.at[0])
        for k in range(zrows // CH):
            pltpu.async_copy(
                rows.at[0], acc.at[pl.ds(sid * zrows + k * CH, CH)], w0)
        for k in range(zrows // CH):
            pltpu.make_async_copy(
                rows.at[0], acc.at[pl.ds(sid * zrows + k * CH, CH)], w0).wait()
        plsc.subcore_barrier()

        ntr = base_trips + jnp.where(wid < rem, 1, 0).astype(jnp.int32)

        def is_copy(i):
            off = (wid + i * NW) * CH
            return pltpu.make_async_copy(
                src_hbm.at[pl.ds(off, CH)], sib.at[lax.rem(i, 4)], isem)

        def id_copy(i):
            off = (wid + i * NW) * CH
            return pltpu.make_async_copy(
                dst_hbm.at[pl.ds(off, CH)], dib.at[lax.rem(i, 4)], isem)

        def g_copy(i):
            return pltpu.make_async_copy(
                table_hbm.at[sib.at[lax.rem(i, 4)]],
                rows.at[lax.rem(i, 2)], gsem)

        @pl.when(ntr > 0)
        def _():
            is_copy(0).start()
            id_copy(0).start()
            is_copy(0).wait()
            id_copy(0).wait()
            g_copy(0).start()

            @pl.when(ntr > 1)
            def _():
                is_copy(1).start()
                id_copy(1).start()

        def body(i, carry):
            @pl.when(i + 1 < ntr)
            def _():
                is_copy(i + 1).wait()
                id_copy(i + 1).wait()

            @pl.when(i + 2 < ntr)
            def _():
                is_copy(i + 2).start()
                id_copy(i + 2).start()

            g_copy(i).wait()

            @pl.when(i + 1 < ntr)
            def _():
                g_copy(i + 1).start()

            pltpu.sync_copy(rows.at[lax.rem(i, 2)],
                            acc.at[dib.at[lax.rem(i, 4)]], add=True)
            return carry

        lax.fori_loop(0, ntr, body, 0)
        plsc.subcore_barrier()

        nwb = zrows // CH
        for k in range(nwb):
            s = k % 2
            r0 = sid * zrows + k * CH
            if k >= 2:
                rp = sid * zrows + (k - 2) * CH
                pltpu.make_async_copy(
                    rows.at[s], out_hbm.at[cid, pl.ds(rp, CH)], wsem[s]).wait()
            pltpu.sync_copy(acc.at[pl.ds(r0, CH)], rows.at[s])
            pltpu.async_copy(rows.at[s], out_hbm.at[cid, pl.ds(r0, CH)], wsem[s])
        for k in range(nwb - 2, nwb):
            s = k % 2
            r0 = sid * zrows + k * CH
            pltpu.make_async_copy(
                rows.at[s], out_hbm.at[cid, pl.ds(r0, CH)], wsem[s]).wait()

    return agg(table, src_r, dst_r, zeros_blk)


def _tc_layer(agg2, hprev, wt, wlt, bias, apply_elu, n):
    bn = 1000
    dout = wt.shape[1]

    def body(p0_r, p1_r, hp_r, wt_r, wlt_r, b_r, o_r):
        aggm = p0_r[0] + p1_r[0]
        y = jnp.dot(aggm, wt_r[...], preferred_element_type=jnp.float32)
        y = y + jnp.dot(hp_r[...], wlt_r[...], preferred_element_type=jnp.float32)
        y = y + b_r[...]
        if apply_elu:
            y = jnp.where(y > 0, y, jnp.exp(jnp.minimum(y, 0.0)) - 1.0)
        o_r[...] = y

    return pl.pallas_call(
        body,
        grid=(n // bn,),
        in_specs=[
            pl.BlockSpec((1, bn, D), lambda i: (0, i, 0)),
            pl.BlockSpec((1, bn, D), lambda i: (1, i, 0)),
            pl.BlockSpec((bn, D), lambda i: (i, 0)),
            pl.BlockSpec((D, dout), lambda i: (0, 0)),
            pl.BlockSpec((D, dout), lambda i: (0, 0)),
            pl.BlockSpec((1, dout), lambda i: (0, 0)),
        ],
        out_specs=pl.BlockSpec((bn, dout), lambda i: (i, 0)),
        out_shape=jax.ShapeDtypeStruct((n, dout), jnp.float32),
    )(agg2, agg2, hprev, wt, wlt, bias)


def kernel(x, edge_index, W1, b1, W2, b2, W3, b3,
           Wl1, bl1, Wl2, bl2, Wl3, bl3):
    n = x.shape[0]
    src_r = edge_index[0]
    dst_r = edge_index[1]
    zeros_blk = jnp.zeros((CH, D), jnp.float32)

    agg1 = _sc_aggregate(x, src_r, dst_r, zeros_blk)
    h1 = _tc_layer(agg1, x, W1.T, Wl1.T, (b1 + bl1)[None, :], True, n)
    agg2 = _sc_aggregate(h1, src_r, dst_r, zeros_blk)
    h2 = _tc_layer(agg2, h1, W2.T, Wl2.T, (b2 + bl2)[None, :], True, n)
    agg3 = _sc_aggregate(h2, src_r, dst_r, zeros_blk)
    w3p = jnp.pad(W3, ((0, D - W3.shape[0]), (0, 0)))
    wl3p = jnp.pad(Wl3, ((0, D - Wl3.shape[0]), (0, 0)))
    b3p = jnp.pad(b3 + bl3, (0, D - b3.shape[0]))
    out = _tc_layer(agg3, h2, w3p.T, wl3p.T, b3p[None, :], False, n)
    return out[:, :W3.shape[0]]


# speedup vs baseline: 3.6087x; 1.2255x over previous
import functools

import jax
import jax.numpy as jnp
from jax import lax
from jax.experimental import pallas as pl
from jax.experimental.pallas import tpu as pltpu
from jax.experimental.pallas import tpu_sc as plsc

D = 128
CH = 128
NC = 2
NS = 16
NW = NC * NS
ACC_ROWS = 10112


def _sc_aggregate(table, src_r, dst_r, zeros_blk):
    nch = src_r.shape[0] // CH
    base_trips, rem = divmod(nch, NW)
    zrows = ACC_ROWS // NS

    mesh = plsc.VectorSubcoreMesh(
        core_axis_name="c", subcore_axis_name="s",
        num_cores=NC, num_subcores=NS)

    @functools.partial(
        pl.kernel,
        out_type=jax.ShapeDtypeStruct((NC, ACC_ROWS, D), jnp.float32),
        mesh=mesh,
        scratch_types=[
            pltpu.VMEM_SHARED((ACC_ROWS, D), jnp.float32),
            pltpu.VMEM((3, CH, D), jnp.float32),
            pltpu.VMEM((3, CH), jnp.int32),
            pltpu.VMEM((4, CH), jnp.int32),
            pltpu.SemaphoreType.DMA,
            pltpu.SemaphoreType.DMA,
            pltpu.SemaphoreType.DMA,
            pltpu.SemaphoreType.DMA,
            pltpu.SemaphoreType.DMA,
        ],
    )
    def agg(table_hbm, src_hbm, dst_hbm, zeros_hbm, out_hbm,
            acc, rows, sib, dib, isem, g0, g1, g2, wsem):
        gsem = (g0, g1, g2)
        cid = lax.axis_index("c")
        sid = lax.axis_index("s")
        wid = sid * NC + cid

        zchunks = [(k * CH, CH) for k in range(zrows // CH)]
        tail = zrows - (zrows // CH) * CH
        if tail:
            zchunks.append(((zrows // CH) * CH, tail))

        for (r0, sz) in zchunks:
            pltpu.async_copy(zeros_hbm.at[pl.ds(0, sz)],
                             acc.at[pl.ds(sid * zrows + r0, sz)], wsem)
        for (r0, sz) in zchunks:
            pltpu.make_async_copy(
                zeros_hbm.at[pl.ds(0, sz)],
                acc.at[pl.ds(sid * zrows + r0, sz)], wsem).wait()
        plsc.subcore_barrier()

        ntr = base_trips + jnp.where(wid < rem, 1, 0).astype(jnp.int32)

        def is_copy(i):
            off = (wid + i * NW) * CH
            return pltpu.make_async_copy(
                src_hbm.at[pl.ds(off, CH)], sib.at[lax.rem(i, 3)], isem)

        def id_copy(i):
            off = (wid + i * NW) * CH
            return pltpu.make_async_copy(
                dst_hbm.at[pl.ds(off, CH)], dib.at[lax.rem(i, 4)], isem)

        def g_copy(i, s):
            return pltpu.make_async_copy(
                table_hbm.at[sib.at[lax.rem(i, 3)]],
                rows.at[lax.rem(i, 3)], gsem[s])

        for k in range(2):
            @pl.when(ntr > k)
            def _(k=k):
                is_copy(k).start()
                id_copy(k).start()
                is_copy(k).wait()
                id_copy(k).wait()
                g_copy(k, k).start()

        @pl.when(ntr > 2)
        def _():
            is_copy(2).start()
            id_copy(2).start()

        def chunk_step(i, s):
            g_copy(i, s).wait()
            j = i + 2

            @pl.when(j < ntr)
            def _():
                is_copy(j).wait()
                id_copy(j).wait()

                @pl.when(i + 3 < ntr)
                def _():
                    is_copy(i + 3).start()
                    id_copy(i + 3).start()
                g_copy(j, (s + 2) % 3).start()

            pltpu.sync_copy(rows.at[lax.rem(i, 3)],
                            acc.at[dib.at[lax.rem(i, 4)]], add=True)

        def body(t, carry):
            for u in range(3):
                i = t * 3 + u

                @pl.when(i < ntr)
                def _(i=i, u=u):
                    chunk_step(i, u)
            return carry

        lax.fori_loop(0, (ntr + 2) // 3, body, 0)
        plsc.subcore_barrier()

        for (r0, sz) in zchunks:
            ra = sid * zrows + r0
            pltpu.async_copy(acc.at[pl.ds(ra, sz)],
                             out_hbm.at[cid, pl.ds(ra, sz)], wsem)
        for (r0, sz) in zchunks:
            ra = sid * zrows + r0
            pltpu.make_async_copy(
                acc.at[pl.ds(ra, sz)],
                out_hbm.at[cid, pl.ds(ra, sz)], wsem).wait()

    return agg(table, src_r, dst_r, zeros_blk)


def _tc_layer(agg2, hprev, wt, wlt, bias, apply_elu, n):
    bn = 1000
    dout = wt.shape[1]

    def body(p0_r, p1_r, hp_r, wt_r, wlt_r, b_r, o_r):
        aggm = p0_r[0] + p1_r[0]
        y = jnp.dot(aggm, wt_r[...], preferred_element_type=jnp.float32)
        y = y + jnp.dot(hp_r[...], wlt_r[...], preferred_element_type=jnp.float32)
        y = y + b_r[...]
        if apply_elu:
            y = jnp.where(y > 0, y, jnp.exp(jnp.minimum(y, 0.0)) - 1.0)
        o_r[...] = y

    return pl.pallas_call(
        body,
        grid=(n // bn,),
        in_specs=[
            pl.BlockSpec((1, bn, D), lambda i: (0, i, 0)),
            pl.BlockSpec((1, bn, D), lambda i: (1, i, 0)),
            pl.BlockSpec((bn, D), lambda i: (i, 0)),
            pl.BlockSpec((D, dout), lambda i: (0, 0)),
            pl.BlockSpec((D, dout), lambda i: (0, 0)),
            pl.BlockSpec((1, dout), lambda i: (0, 0)),
        ],
        out_specs=pl.BlockSpec((bn, dout), lambda i: (i, 0)),
        out_shape=jax.ShapeDtypeStruct((n, dout), jnp.float32),
    )(agg2, agg2, hprev, wt, wlt, bias)


def kernel(x, edge_index, W1, b1, W2, b2, W3, b3,
           Wl1, bl1, Wl2, bl2, Wl3, bl3):
    n = x.shape[0]
    src_r = edge_index[0]
    dst_r = edge_index[1]
    zeros_blk = jnp.zeros((CH, D), jnp.float32)

    agg1 = _sc_aggregate(x, src_r, dst_r, zeros_blk)
    h1 = _tc_layer(agg1, x, W1.T, Wl1.T, (b1 + bl1)[None, :], True, n)
    agg2 = _sc_aggregate(h1, src_r, dst_r, zeros_blk)
    h2 = _tc_layer(agg2, h1, W2.T, Wl2.T, (b2 + bl2)[None, :], True, n)
    agg3 = _sc_aggregate(h2, src_r, dst_r, zeros_blk)
    w3p = jnp.pad(W3, ((0, D - W3.shape[0]), (0, 0)))
    wl3p = jnp.pad(Wl3, ((0, D - Wl3.shape[0]), (0, 0)))
    b3p = jnp.pad(b3 + bl3, (0, D - b3.shape[0]))
    out = _tc_layer(agg3, h2, w3p.T, wl3p.T, b3p[None, :], False, n)
    return out[:, :W3.shape[0]]


# speedup vs baseline: 3.6507x; 1.0116x over previous
import functools

import jax
import jax.numpy as jnp
from jax import lax
from jax.experimental import pallas as pl
from jax.experimental.pallas import tpu as pltpu
from jax.experimental.pallas import tpu_sc as plsc

D = 128
CH = 128
NC = 2
NS = 16
NW = NC * NS
ACC_ROWS = 10112


def _sc_aggregate(table, eb, zeros_blk):
    nch = eb.shape[0]
    base_trips, rem = divmod(nch, NW)
    zrows = ACC_ROWS // NS

    mesh = plsc.VectorSubcoreMesh(
        core_axis_name="c", subcore_axis_name="s",
        num_cores=NC, num_subcores=NS)

    @functools.partial(
        pl.kernel,
        out_type=jax.ShapeDtypeStruct((NC, ACC_ROWS, D), jnp.float32),
        mesh=mesh,
        scratch_types=[
            pltpu.VMEM_SHARED((ACC_ROWS, D), jnp.float32),
            pltpu.VMEM((3, CH, D), jnp.float32),
            pltpu.VMEM((4, 2, CH), jnp.int32),
            pltpu.SemaphoreType.DMA,
            pltpu.SemaphoreType.DMA,
            pltpu.SemaphoreType.DMA,
            pltpu.SemaphoreType.DMA,
            pltpu.SemaphoreType.DMA,
            pltpu.SemaphoreType.DMA,
            pltpu.SemaphoreType.DMA,
            pltpu.SemaphoreType.DMA,
        ],
    )
    def agg(table_hbm, eb_hbm, zeros_hbm, out_hbm,
            acc, rows, ib, isem, g0, g1, g2, s0, s1, s2, wsem):
        gsem = (g0, g1, g2)
        ssem = (s0, s1, s2)
        cid = lax.axis_index("c")
        sid = lax.axis_index("s")
        wid = sid * NC + cid

        zchunks = [(k * CH, CH) for k in range(zrows // CH)]
        tail = zrows - (zrows // CH) * CH
        if tail:
            zchunks.append(((zrows // CH) * CH, tail))

        ntr = base_trips + jnp.where(wid < rem, 1, 0).astype(jnp.int32)

        def i_copy(i):
            return pltpu.make_async_copy(
                eb_hbm.at[wid + i * NW], ib.at[lax.rem(i, 4)], isem)

        def g_copy(i, s):
            return pltpu.make_async_copy(
                table_hbm.at[ib.at[lax.rem(i, 4), 0]],
                rows.at[lax.rem(i, 3)], gsem[s])

        def s_start(i, s):
            pltpu.async_copy(
                rows.at[lax.rem(i, 3)], acc.at[ib.at[lax.rem(i, 4), 1]],
                ssem[s], add=True)

        def s_wait(i, s):
            pltpu.make_async_copy(
                rows.at[lax.rem(i, 3)], acc.at[ib.at[lax.rem(i, 4), 1]],
                ssem[s]).wait()

        for (r0, sz) in zchunks:
            pltpu.async_copy(zeros_hbm.at[pl.ds(0, sz)],
                             acc.at[pl.ds(sid * zrows + r0, sz)], wsem)
        for k in range(2):
            @pl.when(ntr > k)
            def _(k=k):
                i_copy(k).start()
                i_copy(k).wait()
                g_copy(k, k).start()

        @pl.when(ntr > 2)
        def _():
            i_copy(2).start()
        for (r0, sz) in zchunks:
            pltpu.make_async_copy(
                zeros_hbm.at[pl.ds(0, sz)],
                acc.at[pl.ds(sid * zrows + r0, sz)], wsem).wait()
        plsc.subcore_barrier()

        def chunk_step(i, u):
            g_copy(i, u).wait()
            j = i + 2

            @pl.when(j < ntr)
            def _():
                i_copy(j).wait()

            @pl.when(i >= 1)
            def _():
                s_wait(i - 1, (u + 2) % 3)

            @pl.when(i + 3 < ntr)
            def _():
                i_copy(i + 3).start()

            @pl.when(j < ntr)
            def _():
                g_copy(j, (u + 2) % 3).start()

            s_start(i, u)

        def body(t, carry):
            for u in range(3):
                i = t * 3 + u

                @pl.when(i < ntr)
                def _(i=i, u=u):
                    chunk_step(i, u)
            return carry

        lax.fori_loop(0, (ntr + 2) // 3, body, 0)
        for u in range(3):
            @pl.when(lax.rem(ntr - 1, 3) == u)
            def _(u=u):
                s_wait(ntr - 1, u)
        plsc.subcore_barrier()

        for (r0, sz) in zchunks:
            ra = sid * zrows + r0
            pltpu.async_copy(acc.at[pl.ds(ra, sz)],
                             out_hbm.at[cid, pl.ds(ra, sz)], wsem)
        for (r0, sz) in zchunks:
            ra = sid * zrows + r0
            pltpu.make_async_copy(
                acc.at[pl.ds(ra, sz)],
                out_hbm.at[cid, pl.ds(ra, sz)], wsem).wait()

    return agg(table, eb, zeros_blk)


def _tc_layer(agg2, hprev, wt, wlt, bias, apply_elu, n):
    bn = 1000
    dout = wt.shape[1]

    def body(p0_r, p1_r, hp_r, wt_r, wlt_r, b_r, o_r):
        aggm = p0_r[0] + p1_r[0]
        y = jnp.dot(aggm, wt_r[...], preferred_element_type=jnp.float32)
        y = y + jnp.dot(hp_r[...], wlt_r[...], preferred_element_type=jnp.float32)
        y = y + b_r[...]
        if apply_elu:
            y = jnp.where(y > 0, y, jnp.exp(jnp.minimum(y, 0.0)) - 1.0)
        o_r[...] = y

    return pl.pallas_call(
        body,
        grid=(n // bn,),
        in_specs=[
            pl.BlockSpec((1, bn, D), lambda i: (0, i, 0)),
            pl.BlockSpec((1, bn, D), lambda i: (1, i, 0)),
            pl.BlockSpec((bn, D), lambda i: (i, 0)),
            pl.BlockSpec((D, dout), lambda i: (0, 0)),
            pl.BlockSpec((D, dout), lambda i: (0, 0)),
            pl.BlockSpec((1, dout), lambda i: (0, 0)),
        ],
        out_specs=pl.BlockSpec((bn, dout), lambda i: (i, 0)),
        out_shape=jax.ShapeDtypeStruct((n, dout), jnp.float32),
    )(agg2, agg2, hprev, wt, wlt, bias)


def kernel(x, edge_index, W1, b1, W2, b2, W3, b3,
           Wl1, bl1, Wl2, bl2, Wl3, bl3):
    n = x.shape[0]
    e = edge_index.shape[1]
    eb = edge_index.reshape(2, e // CH, CH).transpose(1, 0, 2)
    zeros_blk = jnp.zeros((CH, D), jnp.float32)

    agg1 = _sc_aggregate(x, eb, zeros_blk)
    h1 = _tc_layer(agg1, x, W1.T, Wl1.T, (b1 + bl1)[None, :], True, n)
    agg2 = _sc_aggregate(h1, eb, zeros_blk)
    h2 = _tc_layer(agg2, h1, W2.T, Wl2.T, (b2 + bl2)[None, :], True, n)
    agg3 = _sc_aggregate(h2, eb, zeros_blk)
    w3p = jnp.pad(W3, ((0, D - W3.shape[0]), (0, 0)))
    wl3p = jnp.pad(Wl3, ((0, D - Wl3.shape[0]), (0, 0)))
    b3p = jnp.pad(b3 + bl3, (0, D - b3.shape[0]))
    out = _tc_layer(agg3, h2, w3p.T, wl3p.T, b3p[None, :], False, n)
    return out[:, :W3.shape[0]]


# speedup vs baseline: 3.7654x; 1.0314x over previous
import functools

import jax
import jax.numpy as jnp
from jax import lax
from jax.experimental import pallas as pl
from jax.experimental.pallas import tpu as pltpu
from jax.experimental.pallas import tpu_sc as plsc

D = 128
CH = 128
NC = 2
NS = 16
NW = NC * NS
ACC_ROWS = 10112


def _sc_aggregate(table, eb, zeros_blk):
    nch = eb.shape[0]
    base_trips, rem = divmod(nch, NW)
    zrows = ACC_ROWS // NS

    mesh = plsc.VectorSubcoreMesh(
        core_axis_name="c", subcore_axis_name="s",
        num_cores=NC, num_subcores=NS)

    @functools.partial(
        pl.kernel,
        out_type=jax.ShapeDtypeStruct((NC, ACC_ROWS, D), jnp.float32),
        mesh=mesh,
        scratch_types=[
            pltpu.VMEM_SHARED((ACC_ROWS, D), jnp.float32),
            pltpu.VMEM((3, CH, D), jnp.float32),
            pltpu.VMEM((4, 2, CH), jnp.int32),
            pltpu.SemaphoreType.DMA,
            pltpu.SemaphoreType.DMA,
            pltpu.SemaphoreType.DMA,
            pltpu.SemaphoreType.DMA,
            pltpu.SemaphoreType.DMA,
            pltpu.SemaphoreType.DMA,
            pltpu.SemaphoreType.DMA,
            pltpu.SemaphoreType.DMA,
        ],
    )
    def agg(table_hbm, eb_hbm, zeros_hbm, out_hbm,
            acc, rows, ib, isem, g0, g1, g2, s0, s1, s2, wsem):
        gsem = (g0, g1, g2)
        ssem = (s0, s1, s2)
        cid = lax.axis_index("c")
        sid = lax.axis_index("s")
        wid = sid * NC + cid

        zchunks = [(k * CH, CH) for k in range(zrows // CH)]
        tail = zrows - (zrows // CH) * CH
        if tail:
            zchunks.append(((zrows // CH) * CH, tail))

        ntr = base_trips + jnp.where(wid < rem, 1, 0).astype(jnp.int32)

        def i_copy(i):
            return pltpu.make_async_copy(
                eb_hbm.at[wid + i * NW], ib.at[lax.rem(i, 4)], isem)

        def g_copy(i, s):
            return pltpu.make_async_copy(
                table_hbm.at[ib.at[lax.rem(i, 4), 0]],
                rows.at[lax.rem(i, 3)], gsem[s])

        def s_start(i, s):
            pltpu.async_copy(
                rows.at[lax.rem(i, 3)], acc.at[ib.at[lax.rem(i, 4), 1]],
                ssem[s], add=True)

        def s_wait(i, s):
            pltpu.make_async_copy(
                rows.at[lax.rem(i, 3)], acc.at[ib.at[lax.rem(i, 4), 1]],
                ssem[s]).wait()

        for (r0, sz) in zchunks:
            pltpu.async_copy(zeros_hbm.at[pl.ds(0, sz)],
                             acc.at[pl.ds(sid * zrows + r0, sz)], wsem)
        for k in range(2):
            @pl.when(ntr > k)
            def _(k=k):
                i_copy(k).start()
                i_copy(k).wait()
                g_copy(k, k).start()

        @pl.when(ntr > 2)
        def _():
            i_copy(2).start()
        for (r0, sz) in zchunks:
            pltpu.make_async_copy(
                zeros_hbm.at[pl.ds(0, sz)],
                acc.at[pl.ds(sid * zrows + r0, sz)], wsem).wait()
        plsc.subcore_barrier()

        def chunk_step(i, u):
            g_copy(i, u).wait()
            s_start(i, u)
            j = i + 2

            @pl.when(j < ntr)
            def _():
                i_copy(j).wait()

            @pl.when(i >= 1)
            def _():
                s_wait(i - 1, (u + 2) % 3)

            @pl.when(i + 3 < ntr)
            def _():
                i_copy(i + 3).start()

            @pl.when(j < ntr)
            def _():
                g_copy(j, (u + 2) % 3).start()

        def body(t, carry):
            for u in range(3):
                i = t * 3 + u

                @pl.when(i < ntr)
                def _(i=i, u=u):
                    chunk_step(i, u)
            return carry

        lax.fori_loop(0, (ntr + 2) // 3, body, 0)
        for u in range(3):
            @pl.when(lax.rem(ntr - 1, 3) == u)
            def _(u=u):
                s_wait(ntr - 1, u)
        plsc.subcore_barrier()

        for (r0, sz) in zchunks:
            ra = sid * zrows + r0
            pltpu.async_copy(acc.at[pl.ds(ra, sz)],
                             out_hbm.at[cid, pl.ds(ra, sz)], wsem)
        for (r0, sz) in zchunks:
            ra = sid * zrows + r0
            pltpu.make_async_copy(
                acc.at[pl.ds(ra, sz)],
                out_hbm.at[cid, pl.ds(ra, sz)], wsem).wait()

    return agg(table, eb, zeros_blk)


def _tc_layer(agg2, hprev, wt, wlt, bias, apply_elu, n):
    bn = 2000
    dout = wt.shape[1]

    def body(p0_r, p1_r, hp_r, wt_r, wlt_r, b_r, o_r):
        aggm = p0_r[0] + p1_r[0]
        y = jnp.dot(aggm, wt_r[...], preferred_element_type=jnp.float32)
        y = y + jnp.dot(hp_r[...], wlt_r[...], preferred_element_type=jnp.float32)
        y = y + b_r[...]
        if apply_elu:
            y = jnp.where(y > 0, y, jnp.exp(jnp.minimum(y, 0.0)) - 1.0)
        o_r[...] = y

    return pl.pallas_call(
        body,
        grid=(n // bn,),
        in_specs=[
            pl.BlockSpec((1, bn, D), lambda i: (0, i, 0)),
            pl.BlockSpec((1, bn, D), lambda i: (1, i, 0)),
            pl.BlockSpec((bn, D), lambda i: (i, 0)),
            pl.BlockSpec((D, dout), lambda i: (0, 0)),
            pl.BlockSpec((D, dout), lambda i: (0, 0)),
            pl.BlockSpec((1, dout), lambda i: (0, 0)),
        ],
        out_specs=pl.BlockSpec((bn, dout), lambda i: (i, 0)),
        out_shape=jax.ShapeDtypeStruct((n, dout), jnp.float32),
    )(agg2, agg2, hprev, wt, wlt, bias)


def kernel(x, edge_index, W1, b1, W2, b2, W3, b3,
           Wl1, bl1, Wl2, bl2, Wl3, bl3):
    n = x.shape[0]
    e = edge_index.shape[1]
    eb = edge_index.reshape(2, e // CH, CH).transpose(1, 0, 2)
    zeros_blk = jnp.zeros((CH, D), jnp.float32)

    agg1 = _sc_aggregate(x, eb, zeros_blk)
    h1 = _tc_layer(agg1, x, W1.T, Wl1.T, (b1 + bl1)[None, :], True, n)
    agg2 = _sc_aggregate(h1, eb, zeros_blk)
    h2 = _tc_layer(agg2, h1, W2.T, Wl2.T, (b2 + bl2)[None, :], True, n)
    agg3 = _sc_aggregate(h2, eb, zeros_blk)
    return _tc_layer(agg3, h2, W3.T, Wl3.T, (b3 + bl3)[None, :], False, n)
